# baseline (speedup 1.0000x reference)
import functools

import jax
import jax.numpy as jnp
from jax import lax
from jax.experimental import pallas as pl
from jax.experimental.pallas import tpu as pltpu
from jax.experimental.pallas import tpu_sc as plsc

N = 10000
E = 320000
DIN = 128
DE = 16
DOUT = 128
DEDGE = 256

NC = 2
NS = 16
NW = NC * NS

CHUNK = 128
NCHUNK = E // CHUNK
BASE_K = NCHUNK // NW
EXTRA_W = NCHUNK - BASE_K * NW

STRIPE = 624
LAST_STRIPE = N - 15 * STRIPE

_mesh = plsc.VectorSubcoreMesh(core_axis_name="c", subcore_axis_name="s")


def _worker_id():
    return lax.axis_index("s") * NC + lax.axis_index("c")


def _num_chunks(wid):
    return BASE_K + jnp.where(wid < EXTRA_W, 1, 0)


WPT = N * DE // NS


@functools.partial(
    pl.kernel,
    out_type=(
        jax.ShapeDtypeStruct((N * DE,), jnp.float32),
        jax.ShapeDtypeStruct((N * DE,), jnp.float32),
        jax.ShapeDtypeStruct((N,), jnp.float32),
        jax.ShapeDtypeStruct((N,), jnp.float32),
    ),
    mesh=_mesh,
    scratch_types=[
        pltpu.VMEM_SHARED((N * DE,), jnp.float32),
        pltpu.VMEM_SHARED((N,), jnp.float32),
        pltpu.VMEM((DE, CHUNK), jnp.float32),
        pltpu.VMEM((DE, CHUNK), jnp.int32),
        pltpu.VMEM((CHUNK,), jnp.int32),
        pltpu.VMEM((CHUNK,), jnp.float32),
        pltpu.VMEM((WPT,), jnp.float32),
        pltpu.SemaphoreType.DMA,
        pltpu.SemaphoreType.DMA,
    ],
)
def _sc_segment(eft_hbm, idx_hbm, v_hbm, msum0_out, msum1_out, cnt0_out,
                cnt1_out, msum_sp, cnt_sp, colbuf, idxbuf, vbuf, onesbuf,
                zflat, sin, ssc):
    cid = lax.axis_index("c")
    sid = lax.axis_index("s")
    wid = _worker_id()

    zeros16 = jnp.zeros((16,), jnp.float32)
    for i in range(CHUNK // 16):
        onesbuf[pl.ds(i * 16, 16)] = jnp.full((16,), 1.0, jnp.float32)

    def zrow(r, _):
        zflat[pl.ds(r * 16, 16)] = zeros16
        return _

    lax.fori_loop(0, WPT // 16, zrow, 0)

    pltpu.sync_copy(zflat, msum_sp.at[pl.ds(sid * WPT, WPT)])

    @pl.when(sid < 15)
    def _():
        pltpu.sync_copy(zflat.at[pl.ds(0, STRIPE)],
                        cnt_sp.at[pl.ds(sid * STRIPE, STRIPE)])

    @pl.when(sid == 15)
    def _():
        pltpu.sync_copy(zflat.at[pl.ds(0, LAST_STRIPE)],
                        cnt_sp.at[pl.ds(15 * STRIPE, LAST_STRIPE)])

    plsc.subcore_barrier()

    def body(k, _):
        c = wid + k * NW
        base = c * CHUNK
        for col in range(DE):
            pltpu.async_copy(eft_hbm.at[pl.ds(col * E + base, CHUNK)],
                             colbuf.at[col], sin)
            pltpu.async_copy(idx_hbm.at[pl.ds(col * E + base, CHUNK)],
                             idxbuf.at[col], sin)
        pltpu.async_copy(v_hbm.at[pl.ds(base, CHUNK)], vbuf, sin)
        for col in range(DE):
            pltpu.make_async_copy(eft_hbm.at[pl.ds(col * E + base, CHUNK)],
                                  colbuf.at[col], sin).wait()
            pltpu.make_async_copy(idx_hbm.at[pl.ds(col * E + base, CHUNK)],
                                  idxbuf.at[col], sin).wait()
        pltpu.make_async_copy(v_hbm.at[pl.ds(base, CHUNK)], vbuf, sin).wait()
        for col in range(DE):
            pltpu.async_copy(colbuf.at[col], msum_sp.at[idxbuf.at[col]],
                             ssc, add=True)
        pltpu.async_copy(onesbuf, cnt_sp.at[vbuf], ssc, add=True)
        for col in range(DE):
            pltpu.make_async_copy(colbuf.at[col], msum_sp.at[idxbuf.at[col]],
                                  ssc).wait()
        pltpu.make_async_copy(onesbuf, cnt_sp.at[vbuf], ssc).wait()
        return _

    lax.fori_loop(0, _num_chunks(wid), body, 0)

    plsc.subcore_barrier()

    pltpu.sync_copy(msum_sp.at[pl.ds(sid * WPT, WPT)], zflat)

    @pl.when(cid == 0)
    def _():
        pltpu.sync_copy(zflat, msum0_out.at[pl.ds(sid * WPT, WPT)])

    @pl.when(cid == 1)
    def _():
        pltpu.sync_copy(zflat, msum1_out.at[pl.ds(sid * WPT, WPT)])

    @pl.when(sid < 15)
    def _():
        pltpu.sync_copy(cnt_sp.at[pl.ds(sid * STRIPE, STRIPE)],
                        zflat.at[pl.ds(0, STRIPE)])

    @pl.when(sid == 15)
    def _():
        pltpu.sync_copy(cnt_sp.at[pl.ds(15 * STRIPE, LAST_STRIPE)],
                        zflat.at[pl.ds(0, LAST_STRIPE)])

    @pl.when(jnp.logical_and(sid < 15, cid == 0))
    def _():
        pltpu.sync_copy(zflat.at[pl.ds(0, STRIPE)],
                        cnt0_out.at[pl.ds(sid * STRIPE, STRIPE)])

    @pl.when(jnp.logical_and(sid == 15, cid == 0))
    def _():
        pltpu.sync_copy(zflat.at[pl.ds(0, LAST_STRIPE)],
                        cnt0_out.at[pl.ds(15 * STRIPE, LAST_STRIPE)])

    @pl.when(jnp.logical_and(sid < 15, cid == 1))
    def _():
        pltpu.sync_copy(zflat.at[pl.ds(0, STRIPE)],
                        cnt1_out.at[pl.ds(sid * STRIPE, STRIPE)])

    @pl.when(jnp.logical_and(sid == 15, cid == 1))
    def _():
        pltpu.sync_copy(zflat.at[pl.ds(0, LAST_STRIPE)],
                        cnt1_out.at[pl.ds(15 * STRIPE, LAST_STRIPE)])


BE = 2560
NBE = E // BE


def _prep_body(ef_ref, v_ref, eft_ref, idx_ref):
    vb = v_ref[0, 0, :]
    iota = lax.broadcasted_iota(jnp.int32, (DE, BE), 0)
    idx_ref[...] = vb[None, :] * DE + iota
    eft_ref[...] = ef_ref[...].T


def _tc_prep(ef, v):
    return pl.pallas_call(
        _prep_body,
        grid=(NBE,),
        in_specs=[
            pl.BlockSpec((BE, DE), lambda i: (i, 0)),
            pl.BlockSpec((1, 1, BE), lambda i: (i, 0, 0)),
        ],
        out_specs=[
            pl.BlockSpec((DE, BE), lambda i: (0, i)),
            pl.BlockSpec((DE, BE), lambda i: (0, i)),
        ],
        out_shape=[
            jax.ShapeDtypeStruct((DE, E), jnp.float32),
            jax.ShapeDtypeStruct((DE, E), jnp.int32),
        ],
    )(ef, v.reshape(NBE, 1, BE))


NB = 10
BN = N // NB


def _tc_body(nf_ref, m0_ref, m1_ref, c0_ref, c1_ref, wa_ref, ba_ref, we_ref,
             be_ref, h_ref, hpu_ref, hpv_ref):
    msum = m0_ref[...] + m1_ref[...]
    cnt = c0_ref[0, 0, :] + c1_ref[0, 0, :]
    recip = 1.0 / jnp.maximum(cnt, 1.0)
    h_neigh = msum * recip[:, None]
    x = jnp.dot(nf_ref[...], wa_ref[0:DIN, :],
                preferred_element_type=jnp.float32)
    x += jnp.dot(h_neigh, wa_ref[DIN:DIN + DE, :],
                 preferred_element_type=jnp.float32)
    h = jnp.maximum(x + ba_ref[...], 0.0)
    h_ref[...] = h
    hpu_ref[...] = jnp.dot(h, we_ref[0:DOUT, :],
                           preferred_element_type=jnp.float32) + be_ref[...]
    hpv_ref[...] = jnp.dot(h, we_ref[DOUT:2 * DOUT, :],
                           preferred_element_type=jnp.float32)


def _tc_dense(nf, msum0, msum1, cnt0, cnt1, W_apply, b_apply, W_edge, b_edge):
    return pl.pallas_call(
        _tc_body,
        grid=(NB,),
        in_specs=[
            pl.BlockSpec((BN, DIN), lambda i: (i, 0)),
            pl.BlockSpec((BN, DE), lambda i: (i, 0)),
            pl.BlockSpec((BN, DE), lambda i: (i, 0)),
            pl.BlockSpec((1, 1, BN), lambda i: (i, 0, 0)),
            pl.BlockSpec((1, 1, BN), lambda i: (i, 0, 0)),
            pl.BlockSpec((DIN + DE, DOUT), lambda i: (0, 0)),
            pl.BlockSpec((1, DOUT), lambda i: (0, 0)),
            pl.BlockSpec((2 * DOUT, DEDGE), lambda i: (0, 0)),
            pl.BlockSpec((1, DEDGE), lambda i: (0, 0)),
        ],
        out_specs=[
            pl.BlockSpec((BN, DOUT), lambda i: (i, 0)),
            pl.BlockSpec((BN, DEDGE), lambda i: (i, 0)),
            pl.BlockSpec((BN, DEDGE), lambda i: (i, 0)),
        ],
        out_shape=[
            jax.ShapeDtypeStruct((N, DOUT), jnp.float32),
            jax.ShapeDtypeStruct((N, DEDGE), jnp.float32),
            jax.ShapeDtypeStruct((N, DEDGE), jnp.float32),
        ],
    )(nf, msum0.reshape(N, DE), msum1.reshape(N, DE),
      cnt0.reshape(NB, 1, BN), cnt1.reshape(NB, 1, BN),
      W_apply, b_apply, W_edge, b_edge)


DHALF = DEDGE // 2
CH_C = 64
NCH_C = E // CH_C
BASE_C = NCH_C // NW
EXTRA_C = NCH_C - BASE_C * NW
ROW_W = CH_C * DEDGE


@functools.partial(
    pl.kernel,
    out_type=jax.ShapeDtypeStruct((2 * E, DHALF), jnp.float32),
    mesh=_mesh,
    scratch_types=[
        pltpu.VMEM((2, CH_C), jnp.int32),
        pltpu.VMEM((2, CH_C), jnp.int32),
        pltpu.VMEM((CH_C, DHALF), jnp.float32),
        pltpu.VMEM((CH_C, DHALF), jnp.float32),
        pltpu.VMEM((CH_C, DHALF), jnp.float32),
        pltpu.VMEM((CH_C, DHALF), jnp.float32),
        pltpu.VMEM((CH_C, DHALF), jnp.float32),
        pltpu.VMEM((CH_C, DHALF), jnp.float32),
        pltpu.VMEM((CH_C, DHALF), jnp.float32),
        pltpu.VMEM((CH_C, DHALF), jnp.float32),
        pltpu.VMEM((2 * CH_C, DHALF), jnp.float32),
        pltpu.VMEM((2 * CH_C, DHALF), jnp.float32),
        pltpu.SemaphoreType.DMA,
        pltpu.SemaphoreType.DMA,
        pltpu.SemaphoreType.DMA,
        pltpu.SemaphoreType.DMA,
        pltpu.SemaphoreType.DMA,
    ],
)
def _sc_edge(tuA, tvA, tuB, tvB, u_hbm, v_hbm, out_hbm,
             idxb0, idxb1, buA0, bvA0, buB0, bvB0, buA1, bvA1, buB1, bvB1,
             ob0, ob1, sidx, sg0, sg1, so0, so1):
    wid = _worker_id()
    nk = BASE_C + jnp.where(wid < EXTRA_C, 1, 0)
    slots = ((idxb0, buA0, bvA0, buB0, bvB0, ob0, sg0, so0),
             (idxb1, buA1, bvA1, buB1, bvB1, ob1, sg1, so1))

    def load_fire(k, slot):
        idxb, buA, bvA, buB, bvB, ob, sg, so = slot
        base = (wid + k * NW) * CH_C
        pltpu.async_copy(u_hbm.at[pl.ds(base, CH_C)], idxb.at[0], sidx)
        pltpu.async_copy(v_hbm.at[pl.ds(base, CH_C)], idxb.at[1], sidx)
        pltpu.make_async_copy(u_hbm.at[pl.ds(base, CH_C)], idxb.at[0], sidx).wait()
        pltpu.make_async_copy(v_hbm.at[pl.ds(base, CH_C)], idxb.at[1], sidx).wait()
        pltpu.async_copy(tuA.at[idxb.at[0]], buA, sg)
        pltpu.async_copy(tvA.at[idxb.at[1]], bvA, sg)
        pltpu.async_copy(tuB.at[idxb.at[0]], buB, sg)
        pltpu.async_copy(tvB.at[idxb.at[1]], bvB, sg)

    @pl.when(nk > 0)
    def _():
        load_fire(0, slots[0])

    def pair(p, carry):
        for b in (0, 1):
            k = 2 * p + b
            idxb, buA, bvA, buB, bvB, ob, sg, so = slots[b]

            @pl.when(k < nk)
            def _():
                @pl.when(k + 1 < nk)
                def _():
                    load_fire(k + 1, slots[1 - b])

                pltpu.make_async_copy(tuA.at[idxb.at[0]], buA, sg).wait()
                pltpu.make_async_copy(tvA.at[idxb.at[1]], bvA, sg).wait()
                pltpu.make_async_copy(tuB.at[idxb.at[0]], buB, sg).wait()
                pltpu.make_async_copy(tvB.at[idxb.at[1]], bvB, sg).wait()

                @pl.when(k >= 2)
                def _():
                    prev_r = (wid + (k - 2) * NW) * 2 * CH_C
                    pltpu.make_async_copy(
                        ob, out_hbm.at[pl.ds(prev_r, 2 * CH_C)], so).wait()

                def addf(r, _2):
                    for j in range(DHALF // 16):
                        x = buA[r, pl.ds(j * 16, 16)]
                        y = bvA[r, pl.ds(j * 16, 16)]
                        ob[2 * r, pl.ds(j * 16, 16)] = x + y
                        z = buB[r, pl.ds(j * 16, 16)]
                        w = bvB[r, pl.ds(j * 16, 16)]
                        ob[2 * r + 1, pl.ds(j * 16, 16)] = z + w
                    return _2

                lax.fori_loop(0, CH_C, addf, 0)
                base_r = (wid + k * NW) * 2 * CH_C
                pltpu.async_copy(ob, out_hbm.at[pl.ds(base_r, 2 * CH_C)], so)
        return carry

    lax.fori_loop(0, (nk + 1) // 2, pair, 0)

    for b in (0, 1):
        ob = slots[b][5]
        so = slots[b][7]

        @pl.when(nk > b)
        def _():
            kb = ((nk - 1 - b) // 2) * 2 + b
            last_r = (wid + kb * NW) * 2 * CH_C
            pltpu.make_async_copy(
                ob, out_hbm.at[pl.ds(last_r, 2 * CH_C)], so).wait()


def kernel(nfeats, efeats, edge_index, W_apply, b_apply, W_edge, b_edge):
    nf = nfeats.reshape(N, DIN)
    ef = efeats.reshape(E, DE)
    ei = edge_index.astype(jnp.int32)
    u = ei[0]
    v = ei[1]
    eft, idx_all = _tc_prep(ef, v)
    msum0, msum1, cnt0, cnt1 = _sc_segment(
        eft.reshape(DE * E), idx_all.reshape(DE * E), v)
    h, hpu, hpv = _tc_dense(nf, msum0, msum1, cnt0, cnt1, W_apply,
                            b_apply.reshape(1, DOUT), W_edge,
                            b_edge.reshape(1, DEDGE))
    edge = _sc_edge(hpu[:, :DHALF], hpv[:, :DHALF],
                    hpu[:, DHALF:], hpv[:, DHALF:], u, v)
    return h.reshape(N, 1, DOUT), edge.reshape(E, 1, DEDGE)


# speedup vs baseline: 1.0646x; 1.0646x over previous
import functools

import jax
import jax.numpy as jnp
from jax import lax
from jax.experimental import pallas as pl
from jax.experimental.pallas import tpu as pltpu
from jax.experimental.pallas import tpu_sc as plsc

N = 10000
E = 320000
DIN = 128
DE = 16
DOUT = 128
DEDGE = 256

NC = 2
NS = 16
NW = NC * NS

CHUNK = 128
NCHUNK = E // CHUNK
BASE_K = NCHUNK // NW
EXTRA_W = NCHUNK - BASE_K * NW

STRIPE = 624
LAST_STRIPE = N - 15 * STRIPE

_mesh = plsc.VectorSubcoreMesh(core_axis_name="c", subcore_axis_name="s")


def _worker_id():
    return lax.axis_index("s") * NC + lax.axis_index("c")


def _num_chunks(wid):
    return BASE_K + jnp.where(wid < EXTRA_W, 1, 0)


WPT = N * DE // NS


@functools.partial(
    pl.kernel,
    out_type=(
        jax.ShapeDtypeStruct((N * DE,), jnp.float32),
        jax.ShapeDtypeStruct((N * DE,), jnp.float32),
        jax.ShapeDtypeStruct((N,), jnp.float32),
        jax.ShapeDtypeStruct((N,), jnp.float32),
    ),
    mesh=_mesh,
    scratch_types=[
        pltpu.VMEM_SHARED((N * DE,), jnp.float32),
        pltpu.VMEM_SHARED((N,), jnp.float32),
        pltpu.VMEM((DE, CHUNK), jnp.float32),
        pltpu.VMEM((DE, CHUNK), jnp.int32),
        pltpu.VMEM((CHUNK,), jnp.int32),
        pltpu.VMEM((CHUNK,), jnp.float32),
        pltpu.VMEM((WPT,), jnp.float32),
        pltpu.SemaphoreType.DMA,
        pltpu.SemaphoreType.DMA,
    ],
)
def _sc_segment(eft_hbm, idx_hbm, v_hbm, msum0_out, msum1_out, cnt0_out,
                cnt1_out, msum_sp, cnt_sp, colbuf, idxbuf, vbuf, onesbuf,
                zflat, sin, ssc):
    cid = lax.axis_index("c")
    sid = lax.axis_index("s")
    wid = _worker_id()

    zeros16 = jnp.zeros((16,), jnp.float32)
    for i in range(CHUNK // 16):
        onesbuf[pl.ds(i * 16, 16)] = jnp.full((16,), 1.0, jnp.float32)

    def zrow(r, _):
        zflat[pl.ds(r * 16, 16)] = zeros16
        return _

    lax.fori_loop(0, WPT // 16, zrow, 0)

    pltpu.sync_copy(zflat, msum_sp.at[pl.ds(sid * WPT, WPT)])

    @pl.when(sid < 15)
    def _():
        pltpu.sync_copy(zflat.at[pl.ds(0, STRIPE)],
                        cnt_sp.at[pl.ds(sid * STRIPE, STRIPE)])

    @pl.when(sid == 15)
    def _():
        pltpu.sync_copy(zflat.at[pl.ds(0, LAST_STRIPE)],
                        cnt_sp.at[pl.ds(15 * STRIPE, LAST_STRIPE)])

    plsc.subcore_barrier()

    def body(k, _):
        c = wid + k * NW
        base = c * CHUNK
        for col in range(DE):
            pltpu.async_copy(eft_hbm.at[pl.ds(col * E + base, CHUNK)],
                             colbuf.at[col], sin)
            pltpu.async_copy(idx_hbm.at[pl.ds(col * E + base, CHUNK)],
                             idxbuf.at[col], sin)
        pltpu.async_copy(v_hbm.at[pl.ds(base, CHUNK)], vbuf, sin)
        for col in range(DE):
            pltpu.make_async_copy(eft_hbm.at[pl.ds(col * E + base, CHUNK)],
                                  colbuf.at[col], sin).wait()
            pltpu.make_async_copy(idx_hbm.at[pl.ds(col * E + base, CHUNK)],
                                  idxbuf.at[col], sin).wait()
        pltpu.make_async_copy(v_hbm.at[pl.ds(base, CHUNK)], vbuf, sin).wait()
        for col in range(DE):
            pltpu.async_copy(colbuf.at[col], msum_sp.at[idxbuf.at[col]],
                             ssc, add=True)
        pltpu.async_copy(onesbuf, cnt_sp.at[vbuf], ssc, add=True)
        for col in range(DE):
            pltpu.make_async_copy(colbuf.at[col], msum_sp.at[idxbuf.at[col]],
                                  ssc).wait()
        pltpu.make_async_copy(onesbuf, cnt_sp.at[vbuf], ssc).wait()
        return _

    lax.fori_loop(0, _num_chunks(wid), body, 0)

    plsc.subcore_barrier()

    pltpu.sync_copy(msum_sp.at[pl.ds(sid * WPT, WPT)], zflat)

    @pl.when(cid == 0)
    def _():
        pltpu.sync_copy(zflat, msum0_out.at[pl.ds(sid * WPT, WPT)])

    @pl.when(cid == 1)
    def _():
        pltpu.sync_copy(zflat, msum1_out.at[pl.ds(sid * WPT, WPT)])

    @pl.when(sid < 15)
    def _():
        pltpu.sync_copy(cnt_sp.at[pl.ds(sid * STRIPE, STRIPE)],
                        zflat.at[pl.ds(0, STRIPE)])

    @pl.when(sid == 15)
    def _():
        pltpu.sync_copy(cnt_sp.at[pl.ds(15 * STRIPE, LAST_STRIPE)],
                        zflat.at[pl.ds(0, LAST_STRIPE)])

    @pl.when(jnp.logical_and(sid < 15, cid == 0))
    def _():
        pltpu.sync_copy(zflat.at[pl.ds(0, STRIPE)],
                        cnt0_out.at[pl.ds(sid * STRIPE, STRIPE)])

    @pl.when(jnp.logical_and(sid == 15, cid == 0))
    def _():
        pltpu.sync_copy(zflat.at[pl.ds(0, LAST_STRIPE)],
                        cnt0_out.at[pl.ds(15 * STRIPE, LAST_STRIPE)])

    @pl.when(jnp.logical_and(sid < 15, cid == 1))
    def _():
        pltpu.sync_copy(zflat.at[pl.ds(0, STRIPE)],
                        cnt1_out.at[pl.ds(sid * STRIPE, STRIPE)])

    @pl.when(jnp.logical_and(sid == 15, cid == 1))
    def _():
        pltpu.sync_copy(zflat.at[pl.ds(0, LAST_STRIPE)],
                        cnt1_out.at[pl.ds(15 * STRIPE, LAST_STRIPE)])


BE = 2560
NBE = E // BE


def _prep_body(ef_ref, v_ref, eft_ref, idx_ref):
    vb = v_ref[0, 0, :]
    iota = lax.broadcasted_iota(jnp.int32, (DE, BE), 0)
    idx_ref[...] = vb[None, :] * DE + iota
    eft_ref[...] = ef_ref[...].T


def _tc_prep(ef, v):
    return pl.pallas_call(
        _prep_body,
        grid=(NBE,),
        in_specs=[
            pl.BlockSpec((BE, DE), lambda i: (i, 0)),
            pl.BlockSpec((1, 1, BE), lambda i: (i, 0, 0)),
        ],
        out_specs=[
            pl.BlockSpec((DE, BE), lambda i: (0, i)),
            pl.BlockSpec((DE, BE), lambda i: (0, i)),
        ],
        out_shape=[
            jax.ShapeDtypeStruct((DE, E), jnp.float32),
            jax.ShapeDtypeStruct((DE, E), jnp.int32),
        ],
    )(ef, v.reshape(NBE, 1, BE))


NB = 10
BN = N // NB


def _tc_body(nf_ref, m0_ref, m1_ref, c0_ref, c1_ref, wa_ref, ba_ref, we_ref,
             be_ref, h_ref, hpu_ref, hpv_ref):
    msum = m0_ref[...] + m1_ref[...]
    cnt = c0_ref[0, 0, :] + c1_ref[0, 0, :]
    recip = 1.0 / jnp.maximum(cnt, 1.0)
    h_neigh = msum * recip[:, None]
    x = jnp.dot(nf_ref[...], wa_ref[0:DIN, :],
                preferred_element_type=jnp.float32)
    x += jnp.dot(h_neigh, wa_ref[DIN:DIN + DE, :],
                 preferred_element_type=jnp.float32)
    h = jnp.maximum(x + ba_ref[...], 0.0)
    h_ref[...] = h
    hpu_ref[...] = jnp.dot(h, we_ref[0:DOUT, :],
                           preferred_element_type=jnp.float32) + be_ref[...]
    hpv_ref[...] = jnp.dot(h, we_ref[DOUT:2 * DOUT, :],
                           preferred_element_type=jnp.float32)


def _tc_dense(nf, msum0, msum1, cnt0, cnt1, W_apply, b_apply, W_edge, b_edge):
    return pl.pallas_call(
        _tc_body,
        grid=(NB,),
        in_specs=[
            pl.BlockSpec((BN, DIN), lambda i: (i, 0)),
            pl.BlockSpec((BN, DE), lambda i: (i, 0)),
            pl.BlockSpec((BN, DE), lambda i: (i, 0)),
            pl.BlockSpec((1, 1, BN), lambda i: (i, 0, 0)),
            pl.BlockSpec((1, 1, BN), lambda i: (i, 0, 0)),
            pl.BlockSpec((DIN + DE, DOUT), lambda i: (0, 0)),
            pl.BlockSpec((1, DOUT), lambda i: (0, 0)),
            pl.BlockSpec((2 * DOUT, DEDGE), lambda i: (0, 0)),
            pl.BlockSpec((1, DEDGE), lambda i: (0, 0)),
        ],
        out_specs=[
            pl.BlockSpec((BN, DOUT), lambda i: (i, 0)),
            pl.BlockSpec((BN, DEDGE), lambda i: (i, 0)),
            pl.BlockSpec((BN, DEDGE), lambda i: (i, 0)),
        ],
        out_shape=[
            jax.ShapeDtypeStruct((N, DOUT), jnp.float32),
            jax.ShapeDtypeStruct((N, DEDGE), jnp.float32),
            jax.ShapeDtypeStruct((N, DEDGE), jnp.float32),
        ],
    )(nf, msum0.reshape(N, DE), msum1.reshape(N, DE),
      cnt0.reshape(NB, 1, BN), cnt1.reshape(NB, 1, BN),
      W_apply, b_apply, W_edge, b_edge)


DHALF = DEDGE // 2
CH_C = 64
NCH_C = E // CH_C
BASE_C = NCH_C // NW
EXTRA_C = NCH_C - BASE_C * NW
ROW_W = CH_C * DEDGE


@functools.partial(
    pl.kernel,
    out_type=jax.ShapeDtypeStruct((2 * E, DHALF), jnp.float32),
    mesh=_mesh,
    scratch_types=[
        pltpu.VMEM(((BASE_C + 1) * CH_C,), jnp.int32),
        pltpu.VMEM(((BASE_C + 1) * CH_C,), jnp.int32),
        pltpu.VMEM((CH_C, DHALF), jnp.float32),
        pltpu.VMEM((CH_C, DHALF), jnp.float32),
        pltpu.VMEM((CH_C, DHALF), jnp.float32),
        pltpu.VMEM((CH_C, DHALF), jnp.float32),
        pltpu.VMEM((CH_C, DHALF), jnp.float32),
        pltpu.VMEM((CH_C, DHALF), jnp.float32),
        pltpu.VMEM((CH_C, DHALF), jnp.float32),
        pltpu.VMEM((CH_C, DHALF), jnp.float32),
        pltpu.VMEM((2 * CH_C, DHALF), jnp.float32),
        pltpu.VMEM((2 * CH_C, DHALF), jnp.float32),
        pltpu.SemaphoreType.DMA,
        pltpu.SemaphoreType.DMA,
        pltpu.SemaphoreType.DMA,
        pltpu.SemaphoreType.DMA,
        pltpu.SemaphoreType.DMA,
    ],
)
def _sc_edge(tuA, tvA, tuB, tvB, u_hbm, v_hbm, out_hbm,
             idxu_all, idxv_all, buA0, bvA0, buB0, bvB0, buA1, bvA1, buB1,
             bvB1, ob0, ob1, sidx, sg0, sg1, so0, so1):
    wid = _worker_id()
    nk = BASE_C + jnp.where(wid < EXTRA_C, 1, 0)
    estart = (wid * BASE_C + jnp.minimum(wid, EXTRA_C)) * CH_C
    nload = (BASE_C + 1) * CH_C
    pltpu.async_copy(u_hbm.at[pl.ds(estart, nload)], idxu_all, sidx)
    pltpu.async_copy(v_hbm.at[pl.ds(estart, nload)], idxv_all, sidx)
    pltpu.make_async_copy(u_hbm.at[pl.ds(estart, nload)], idxu_all, sidx).wait()
    pltpu.make_async_copy(v_hbm.at[pl.ds(estart, nload)], idxv_all, sidx).wait()

    slots = ((buA0, bvA0, buB0, bvB0, ob0, sg0, so0),
             (buA1, bvA1, buB1, bvB1, ob1, sg1, so1))

    def fire(k, slot):
        buA, bvA, buB, bvB, ob, sg, so = slot
        iu = idxu_all.at[pl.ds(k * CH_C, CH_C)]
        iv = idxv_all.at[pl.ds(k * CH_C, CH_C)]
        pltpu.async_copy(tuA.at[iu], buA, sg)
        pltpu.async_copy(tvA.at[iv], bvA, sg)
        pltpu.async_copy(tuB.at[iu], buB, sg)
        pltpu.async_copy(tvB.at[iv], bvB, sg)

    @pl.when(nk > 0)
    def _():
        fire(0, slots[0])

    def pair(p, carry):
        for b in (0, 1):
            k = 2 * p + b
            buA, bvA, buB, bvB, ob, sg, so = slots[b]

            @pl.when(k < nk)
            def _():
                @pl.when(k + 1 < nk)
                def _():
                    fire(k + 1, slots[1 - b])

                iu = idxu_all.at[pl.ds(k * CH_C, CH_C)]
                iv = idxv_all.at[pl.ds(k * CH_C, CH_C)]
                pltpu.make_async_copy(tuA.at[iu], buA, sg).wait()
                pltpu.make_async_copy(tvA.at[iv], bvA, sg).wait()
                pltpu.make_async_copy(tuB.at[iu], buB, sg).wait()
                pltpu.make_async_copy(tvB.at[iv], bvB, sg).wait()

                @pl.when(k >= 2)
                def _():
                    prev_r = (estart + (k - 2) * CH_C) * 2
                    pltpu.make_async_copy(
                        ob, out_hbm.at[pl.ds(prev_r, 2 * CH_C)], so).wait()

                def addf(r, _2):
                    for j in range(DHALF // 16):
                        x = buA[r, pl.ds(j * 16, 16)]
                        y = bvA[r, pl.ds(j * 16, 16)]
                        ob[2 * r, pl.ds(j * 16, 16)] = x + y
                        z = buB[r, pl.ds(j * 16, 16)]
                        w = bvB[r, pl.ds(j * 16, 16)]
                        ob[2 * r + 1, pl.ds(j * 16, 16)] = z + w
                    return _2

                lax.fori_loop(0, CH_C, addf, 0)
                base_r = (estart + k * CH_C) * 2
                pltpu.async_copy(ob, out_hbm.at[pl.ds(base_r, 2 * CH_C)], so)
        return carry

    lax.fori_loop(0, (nk + 1) // 2, pair, 0)

    for b in (0, 1):
        ob = slots[b][4]
        so = slots[b][6]

        @pl.when(nk > b)
        def _():
            kb = ((nk - 1 - b) // 2) * 2 + b
            last_r = (estart + kb * CH_C) * 2
            pltpu.make_async_copy(
                ob, out_hbm.at[pl.ds(last_r, 2 * CH_C)], so).wait()


def kernel(nfeats, efeats, edge_index, W_apply, b_apply, W_edge, b_edge):
    nf = nfeats.reshape(N, DIN)
    ef = efeats.reshape(E, DE)
    ei = edge_index.astype(jnp.int32)
    u = ei[0]
    v = ei[1]
    eft, idx_all = _tc_prep(ef, v)
    msum0, msum1, cnt0, cnt1 = _sc_segment(
        eft.reshape(DE * E), idx_all.reshape(DE * E), v)
    h, hpu, hpv = _tc_dense(nf, msum0, msum1, cnt0, cnt1, W_apply,
                            b_apply.reshape(1, DOUT), W_edge,
                            b_edge.reshape(1, DEDGE))
    pad = jnp.zeros((CH_C,), jnp.int32)
    edge = _sc_edge(hpu[:, :DHALF], hpv[:, :DHALF],
                    hpu[:, DHALF:], hpv[:, DHALF:],
                    jnp.concatenate([u, pad]), jnp.concatenate([v, pad]))
    return h.reshape(N, 1, DOUT), edge.reshape(E, 1, DEDGE)


# speedup vs baseline: 1.3434x; 1.2619x over previous
import functools

import jax
import jax.numpy as jnp
from jax import lax
from jax.experimental import pallas as pl
from jax.experimental.pallas import tpu as pltpu
from jax.experimental.pallas import tpu_sc as plsc

N = 10000
E = 320000
DIN = 128
DE = 16
DOUT = 128
DEDGE = 256

NC = 2
NS = 16
NW = NC * NS

CHUNK = 128
NCHUNK = E // CHUNK
BASE_K = NCHUNK // NW
EXTRA_W = NCHUNK - BASE_K * NW

STRIPE = 624
LAST_STRIPE = N - 15 * STRIPE

_mesh = plsc.VectorSubcoreMesh(core_axis_name="c", subcore_axis_name="s")


def _worker_id():
    return lax.axis_index("s") * NC + lax.axis_index("c")


def _num_chunks(wid):
    return BASE_K + jnp.where(wid < EXTRA_W, 1, 0)


WPT = N * DE // NS


@functools.partial(
    pl.kernel,
    out_type=(
        jax.ShapeDtypeStruct((N * DE,), jnp.float32),
        jax.ShapeDtypeStruct((N * DE,), jnp.float32),
        jax.ShapeDtypeStruct((N,), jnp.float32),
        jax.ShapeDtypeStruct((N,), jnp.float32),
    ),
    mesh=_mesh,
    scratch_types=[
        pltpu.VMEM_SHARED((N * DE,), jnp.float32),
        pltpu.VMEM_SHARED((N,), jnp.float32),
        pltpu.VMEM((DE, CHUNK), jnp.float32),
        pltpu.VMEM((DE, CHUNK), jnp.int32),
        pltpu.VMEM((CHUNK,), jnp.int32),
        pltpu.VMEM((CHUNK,), jnp.float32),
        pltpu.VMEM((WPT,), jnp.float32),
        pltpu.SemaphoreType.DMA,
        pltpu.SemaphoreType.DMA,
    ],
)
def _sc_segment(eft_hbm, idx_hbm, v_hbm, msum0_out, msum1_out, cnt0_out,
                cnt1_out, msum_sp, cnt_sp, colbuf, idxbuf, vbuf, onesbuf,
                zflat, sin, ssc):
    cid = lax.axis_index("c")
    sid = lax.axis_index("s")
    wid = _worker_id()

    zeros16 = jnp.zeros((16,), jnp.float32)
    for i in range(CHUNK // 16):
        onesbuf[pl.ds(i * 16, 16)] = jnp.full((16,), 1.0, jnp.float32)

    def zrow(r, _):
        zflat[pl.ds(r * 16, 16)] = zeros16
        return _

    lax.fori_loop(0, WPT // 16, zrow, 0)

    pltpu.sync_copy(zflat, msum_sp.at[pl.ds(sid * WPT, WPT)])

    @pl.when(sid < 15)
    def _():
        pltpu.sync_copy(zflat.at[pl.ds(0, STRIPE)],
                        cnt_sp.at[pl.ds(sid * STRIPE, STRIPE)])

    @pl.when(sid == 15)
    def _():
        pltpu.sync_copy(zflat.at[pl.ds(0, LAST_STRIPE)],
                        cnt_sp.at[pl.ds(15 * STRIPE, LAST_STRIPE)])

    plsc.subcore_barrier()

    def body(k, _):
        c = wid + k * NW
        base = c * CHUNK
        for col in range(DE):
            pltpu.async_copy(eft_hbm.at[pl.ds(col * E + base, CHUNK)],
                             colbuf.at[col], sin)
            pltpu.async_copy(idx_hbm.at[pl.ds(col * E + base, CHUNK)],
                             idxbuf.at[col], sin)
        pltpu.async_copy(v_hbm.at[pl.ds(base, CHUNK)], vbuf, sin)
        for col in range(DE):
            pltpu.make_async_copy(eft_hbm.at[pl.ds(col * E + base, CHUNK)],
                                  colbuf.at[col], sin).wait()
            pltpu.make_async_copy(idx_hbm.at[pl.ds(col * E + base, CHUNK)],
                                  idxbuf.at[col], sin).wait()
        pltpu.make_async_copy(v_hbm.at[pl.ds(base, CHUNK)], vbuf, sin).wait()
        for col in range(DE):
            pltpu.async_copy(colbuf.at[col], msum_sp.at[idxbuf.at[col]],
                             ssc, add=True)
        pltpu.async_copy(onesbuf, cnt_sp.at[vbuf], ssc, add=True)
        for col in range(DE):
            pltpu.make_async_copy(colbuf.at[col], msum_sp.at[idxbuf.at[col]],
                                  ssc).wait()
        pltpu.make_async_copy(onesbuf, cnt_sp.at[vbuf], ssc).wait()
        return _

    lax.fori_loop(0, _num_chunks(wid), body, 0)

    plsc.subcore_barrier()

    pltpu.sync_copy(msum_sp.at[pl.ds(sid * WPT, WPT)], zflat)

    @pl.when(cid == 0)
    def _():
        pltpu.sync_copy(zflat, msum0_out.at[pl.ds(sid * WPT, WPT)])

    @pl.when(cid == 1)
    def _():
        pltpu.sync_copy(zflat, msum1_out.at[pl.ds(sid * WPT, WPT)])

    @pl.when(sid < 15)
    def _():
        pltpu.sync_copy(cnt_sp.at[pl.ds(sid * STRIPE, STRIPE)],
                        zflat.at[pl.ds(0, STRIPE)])

    @pl.when(sid == 15)
    def _():
        pltpu.sync_copy(cnt_sp.at[pl.ds(15 * STRIPE, LAST_STRIPE)],
                        zflat.at[pl.ds(0, LAST_STRIPE)])

    @pl.when(jnp.logical_and(sid < 15, cid == 0))
    def _():
        pltpu.sync_copy(zflat.at[pl.ds(0, STRIPE)],
                        cnt0_out.at[pl.ds(sid * STRIPE, STRIPE)])

    @pl.when(jnp.logical_and(sid == 15, cid == 0))
    def _():
        pltpu.sync_copy(zflat.at[pl.ds(0, LAST_STRIPE)],
                        cnt0_out.at[pl.ds(15 * STRIPE, LAST_STRIPE)])

    @pl.when(jnp.logical_and(sid < 15, cid == 1))
    def _():
        pltpu.sync_copy(zflat.at[pl.ds(0, STRIPE)],
                        cnt1_out.at[pl.ds(sid * STRIPE, STRIPE)])

    @pl.when(jnp.logical_and(sid == 15, cid == 1))
    def _():
        pltpu.sync_copy(zflat.at[pl.ds(0, LAST_STRIPE)],
                        cnt1_out.at[pl.ds(15 * STRIPE, LAST_STRIPE)])


BE = 2560
NBE = E // BE


def _prep_body(ef_ref, v_ref, eft_ref, idx_ref):
    vb = v_ref[0, 0, :]
    iota = lax.broadcasted_iota(jnp.int32, (DE, BE), 0)
    idx_ref[...] = vb[None, :] * DE + iota
    eft_ref[...] = ef_ref[...].T


def _tc_prep(ef, v):
    return pl.pallas_call(
        _prep_body,
        grid=(NBE,),
        in_specs=[
            pl.BlockSpec((BE, DE), lambda i: (i, 0)),
            pl.BlockSpec((1, 1, BE), lambda i: (i, 0, 0)),
        ],
        out_specs=[
            pl.BlockSpec((DE, BE), lambda i: (0, i)),
            pl.BlockSpec((DE, BE), lambda i: (0, i)),
        ],
        out_shape=[
            jax.ShapeDtypeStruct((DE, E), jnp.float32),
            jax.ShapeDtypeStruct((DE, E), jnp.int32),
        ],
    )(ef, v.reshape(NBE, 1, BE))


NB = 10
BN = N // NB


def _tc_body(nf_ref, m0_ref, m1_ref, c0_ref, c1_ref, wa_ref, ba_ref, we_ref,
             be_ref, h_ref, hpu_ref, hpv_ref):
    msum = m0_ref[...] + m1_ref[...]
    cnt = c0_ref[0, 0, :] + c1_ref[0, 0, :]
    recip = 1.0 / jnp.maximum(cnt, 1.0)
    h_neigh = msum * recip[:, None]
    x = jnp.dot(nf_ref[...], wa_ref[0:DIN, :],
                preferred_element_type=jnp.float32)
    x += jnp.dot(h_neigh, wa_ref[DIN:DIN + DE, :],
                 preferred_element_type=jnp.float32)
    h = jnp.maximum(x + ba_ref[...], 0.0)
    h_ref[...] = h
    hpu_ref[...] = jnp.dot(h, we_ref[0:DOUT, :],
                           preferred_element_type=jnp.float32) + be_ref[...]
    hpv_ref[...] = jnp.dot(h, we_ref[DOUT:2 * DOUT, :],
                           preferred_element_type=jnp.float32)


def _tc_dense(nf, msum0, msum1, cnt0, cnt1, W_apply, b_apply, W_edge, b_edge):
    return pl.pallas_call(
        _tc_body,
        grid=(NB,),
        in_specs=[
            pl.BlockSpec((BN, DIN), lambda i: (i, 0)),
            pl.BlockSpec((BN, DE), lambda i: (i, 0)),
            pl.BlockSpec((BN, DE), lambda i: (i, 0)),
            pl.BlockSpec((1, 1, BN), lambda i: (i, 0, 0)),
            pl.BlockSpec((1, 1, BN), lambda i: (i, 0, 0)),
            pl.BlockSpec((DIN + DE, DOUT), lambda i: (0, 0)),
            pl.BlockSpec((1, DOUT), lambda i: (0, 0)),
            pl.BlockSpec((2 * DOUT, DEDGE), lambda i: (0, 0)),
            pl.BlockSpec((1, DEDGE), lambda i: (0, 0)),
        ],
        out_specs=[
            pl.BlockSpec((BN, DOUT), lambda i: (i, 0)),
            pl.BlockSpec((BN, DEDGE), lambda i: (i, 0)),
            pl.BlockSpec((BN, DEDGE), lambda i: (i, 0)),
        ],
        out_shape=[
            jax.ShapeDtypeStruct((N, DOUT), jnp.float32),
            jax.ShapeDtypeStruct((N, DEDGE), jnp.float32),
            jax.ShapeDtypeStruct((N, DEDGE), jnp.float32),
        ],
    )(nf, msum0.reshape(N, DE), msum1.reshape(N, DE),
      cnt0.reshape(NB, 1, BN), cnt1.reshape(NB, 1, BN),
      W_apply, b_apply, W_edge, b_edge)


DHALF = DEDGE // 2
IDXL = (BASE_K + 1) * CHUNK


@functools.partial(
    pl.kernel,
    out_type=jax.ShapeDtypeStruct((E, DEDGE), jnp.float32),
    mesh=_mesh,
    scratch_types=[
        pltpu.VMEM((IDXL,), jnp.int32),
        pltpu.VMEM((IDXL,), jnp.int32),
        pltpu.VMEM((CHUNK, DHALF), jnp.float32),
        pltpu.VMEM((CHUNK, DHALF), jnp.float32),
        pltpu.VMEM((CHUNK, DHALF), jnp.float32),
        pltpu.VMEM((CHUNK, DHALF), jnp.float32),
        pltpu.SemaphoreType.DMA,
        pltpu.SemaphoreType.DMA,
        pltpu.SemaphoreType.DMA,
    ],
)
def _sc_edge(tuA, tvA, tuB, tvB, u_hbm, v_hbm, out_hbm,
             idxu_all, idxv_all, bu0, bv0, bu1, bv1, sidx, sg0, sg1):
    wid = _worker_id()
    nk = _num_chunks(wid)
    estart = (wid * BASE_K + jnp.minimum(wid, EXTRA_W)) * CHUNK
    pltpu.async_copy(u_hbm.at[pl.ds(estart, IDXL)], idxu_all, sidx)
    pltpu.async_copy(v_hbm.at[pl.ds(estart, IDXL)], idxv_all, sidx)
    pltpu.make_async_copy(u_hbm.at[pl.ds(estart, IDXL)], idxu_all, sidx).wait()
    pltpu.make_async_copy(v_hbm.at[pl.ds(estart, IDXL)], idxv_all, sidx).wait()

    slots = ((bu0, bv0, sg0), (bu1, bv1, sg1))

    for tu, tv, coff in ((tuA, tvA, 0), (tuB, tvB, DHALF)):

        def fire(k, slot):
            bu, bv, sg = slot
            pltpu.async_copy(tu.at[idxu_all.at[pl.ds(k * CHUNK, CHUNK)]], bu, sg)
            pltpu.async_copy(tv.at[idxv_all.at[pl.ds(k * CHUNK, CHUNK)]], bv, sg)

        @pl.when(nk > 0)
        def _():
            fire(0, slots[0])

        def pair(p, carry):
            for b in (0, 1):
                k = 2 * p + b
                bu, bv, sg = slots[b]

                @pl.when(k < nk)
                def _():
                    @pl.when(k + 1 < nk)
                    def _():
                        fire(k + 1, slots[1 - b])

                    iu = idxu_all.at[pl.ds(k * CHUNK, CHUNK)]
                    iv = idxv_all.at[pl.ds(k * CHUNK, CHUNK)]
                    pltpu.make_async_copy(tu.at[iu], bu, sg).wait()
                    pltpu.make_async_copy(tv.at[iv], bv, sg).wait()

                    def add_row(r, _2):
                        for j in range(DHALF // 16):
                            x = bv[r, pl.ds(j * 16, 16)]
                            plsc.addupdate(bu.at[r, pl.ds(j * 16, 16)], x)
                        return _2

                    lax.fori_loop(0, CHUNK, add_row, 0)
                    base = estart + k * CHUNK
                    pltpu.sync_copy(
                        bu, out_hbm.at[pl.ds(base, CHUNK), pl.ds(coff, DHALF)])
            return carry

        lax.fori_loop(0, (nk + 1) // 2, pair, 0)


def kernel(nfeats, efeats, edge_index, W_apply, b_apply, W_edge, b_edge):
    nf = nfeats.reshape(N, DIN)
    ef = efeats.reshape(E, DE)
    ei = edge_index.astype(jnp.int32)
    u = ei[0]
    v = ei[1]
    eft, idx_all = _tc_prep(ef, v)
    msum0, msum1, cnt0, cnt1 = _sc_segment(
        eft.reshape(DE * E), idx_all.reshape(DE * E), v)
    h, hpu, hpv = _tc_dense(nf, msum0, msum1, cnt0, cnt1, W_apply,
                            b_apply.reshape(1, DOUT), W_edge,
                            b_edge.reshape(1, DEDGE))
    pad = jnp.zeros((CHUNK,), jnp.int32)
    edge = _sc_edge(hpu[:, :DHALF], hpv[:, :DHALF],
                    hpu[:, DHALF:], hpv[:, DHALF:],
                    jnp.concatenate([u, pad]), jnp.concatenate([v, pad]))
    return h.reshape(N, 1, DOUT), edge.reshape(E, 1, DEDGE)
